# baseline (speedup 1.0000x reference)
import jax
import jax.numpy as jnp
from jax.experimental import pallas as pl
from jax.experimental.pallas import tpu as pltpu

_N = 32768
_B = 16
_D = 16
_T = 8192
_NT = _N // _T
_OUT = 256


def _fused_kernel(bounds_ref, seg_ref, pts_ref, W1_ref, b1_ref, W2_ref,
                  b2_ref, W3_ref, b3_ref, Wh_ref, bh_ref, y_ref, acc_ref,
                  cnt_ref):
    i = pl.program_id(0)

    @pl.when(i == 0)
    def _():
        acc_ref[...] = jnp.zeros_like(acc_ref)
        cnt_ref[...] = jnp.zeros_like(cnt_ref)

    x = pts_ref[...]
    h = jnp.dot(x, W1_ref[...], preferred_element_type=jnp.float32) + b1_ref[...]
    h = jnp.maximum(h, 0.0)
    h = jnp.dot(h, W2_ref[...], preferred_element_type=jnp.float32) + b2_ref[...]
    h = jnp.maximum(h, 0.0)
    h = jnp.dot(h, W3_ref[...], preferred_element_type=jnp.float32) + b3_ref[...]
    f = h[:, :_D]

    seg = seg_ref[...]
    onehot = (seg == jax.lax.broadcasted_iota(jnp.int32, (1, _B), 1)
              ).astype(jnp.float32)
    cnt_ref[...] += jnp.sum(onehot, axis=0, keepdims=True)

    lo = bounds_ref[i, 0]
    hi = bounds_ref[i, 1]

    for b in range(_B):
        @pl.when(jnp.logical_and(b >= lo, b <= hi))
        def _(b=b):
            fm = f * onehot[:, b:b + 1]
            c = jax.lax.dot_general(
                fm, f, (((0,), (0,)), ((), ())),
                preferred_element_type=jnp.float32)
            cflat = jnp.concatenate(
                [c[k:k + 1, :] for k in range(_D)], axis=1)
            acc_ref[b:b + 1, :] += cflat

    @pl.when(i == _NT - 1)
    def _():
        maxc = jnp.max(cnt_ref[...])
        cov = acc_ref[...] / maxc
        p = jnp.sign(cov) * jnp.sqrt(jnp.abs(cov) + 1e-12)
        y = jnp.dot(p, Wh_ref[...], preferred_element_type=jnp.float32) + bh_ref[...]
        nrm = jnp.sqrt(jnp.sum(y * y, axis=1, keepdims=True))
        y_ref[...] = y / (nrm + 1e-12)


def kernel(points, segment_ids, W1, b1, W2, b2, W3, b3, Wh, bh):
    seg = segment_ids.astype(jnp.int32)
    bounds = jnp.stack([seg[::_T], seg[_T - 1::_T]], axis=1)
    W3p = jnp.pad(W3, ((0, 0), (0, 64 - _D)))
    b3p = jnp.pad(b3, (0, 64 - _D)).reshape(1, -1)
    out = pl.pallas_call(
        _fused_kernel,
        grid=(_NT,),
        in_specs=[
            pl.BlockSpec(memory_space=pltpu.SMEM),
            pl.BlockSpec((_T, 1), lambda i: (i, 0)),
            pl.BlockSpec((_T, 4), lambda i: (i, 0)),
            pl.BlockSpec((4, 64), lambda i: (0, 0)),
            pl.BlockSpec((1, 64), lambda i: (0, 0)),
            pl.BlockSpec((64, 64), lambda i: (0, 0)),
            pl.BlockSpec((1, 64), lambda i: (0, 0)),
            pl.BlockSpec((64, 64), lambda i: (0, 0)),
            pl.BlockSpec((1, 64), lambda i: (0, 0)),
            pl.BlockSpec((_D * _D, _OUT), lambda i: (0, 0)),
            pl.BlockSpec((1, _OUT), lambda i: (0, 0)),
        ],
        out_specs=pl.BlockSpec((_B, _OUT), lambda i: (0, 0)),
        out_shape=jax.ShapeDtypeStruct((_B, _OUT), jnp.float32),
        scratch_shapes=[
            pltpu.VMEM((_B, _OUT), jnp.float32),
            pltpu.VMEM((1, _B), jnp.float32),
        ],
    )(bounds, seg.reshape(_N, 1), points, W1, b1.reshape(1, -1), W2,
      b2.reshape(1, -1), W3p, b3p, Wh, bh.reshape(1, -1))
    return out


# speedup vs baseline: 1.1107x; 1.1107x over previous
import jax
import jax.numpy as jnp
from jax.experimental import pallas as pl
from jax.experimental.pallas import tpu as pltpu

_N = 32768
_B = 16
_D = 16
_T = 2048
_NT = _N // _T
_OUT = 256


def _fused_kernel(bounds_ref, seg_ref, pts_ref, W1_ref, b1_ref, W2_ref,
                  b2_ref, W3_ref, b3_ref, Wh_ref, bh_ref, y_ref, acc_ref,
                  cnt_ref):
    i = pl.program_id(0)

    @pl.when(i == 0)
    def _():
        acc_ref[...] = jnp.zeros_like(acc_ref)
        cnt_ref[...] = jnp.zeros_like(cnt_ref)

    x = pts_ref[...]
    h = jnp.dot(x, W1_ref[...], preferred_element_type=jnp.float32) + b1_ref[...]
    h = jnp.maximum(h, 0.0)
    h = jnp.dot(h, W2_ref[...], preferred_element_type=jnp.float32) + b2_ref[...]
    h = jnp.maximum(h, 0.0)
    h = jnp.dot(h, W3_ref[...], preferred_element_type=jnp.float32) + b3_ref[...]
    f = h[:, :_D]

    seg = seg_ref[...]
    onehot = (seg == jax.lax.broadcasted_iota(jnp.int32, (1, _B), 1)
              ).astype(jnp.float32)
    cnt_ref[...] += jnp.sum(onehot, axis=0, keepdims=True)

    lo = bounds_ref[i, 0]
    hi = bounds_ref[i, 1]

    for b in range(_B):
        @pl.when(jnp.logical_and(b >= lo, b <= hi))
        def _(b=b):
            fm = f * onehot[:, b:b + 1]
            c = jax.lax.dot_general(
                fm, f, (((0,), (0,)), ((), ())),
                preferred_element_type=jnp.float32)
            cflat = jnp.concatenate(
                [c[k:k + 1, :] for k in range(_D)], axis=1)
            acc_ref[b:b + 1, :] += cflat

    @pl.when(i == _NT - 1)
    def _():
        maxc = jnp.max(cnt_ref[...])
        cov = acc_ref[...] / maxc
        p = jnp.sign(cov) * jnp.sqrt(jnp.abs(cov) + 1e-12)
        y = jnp.dot(p, Wh_ref[...], preferred_element_type=jnp.float32) + bh_ref[...]
        nrm = jnp.sqrt(jnp.sum(y * y, axis=1, keepdims=True))
        y_ref[...] = y / (nrm + 1e-12)


def kernel(points, segment_ids, W1, b1, W2, b2, W3, b3, Wh, bh):
    seg = segment_ids.astype(jnp.int32)
    bounds = jnp.stack([seg[::_T], seg[_T - 1::_T]], axis=1)
    W3p = jnp.pad(W3, ((0, 0), (0, 64 - _D)))
    b3p = jnp.pad(b3, (0, 64 - _D)).reshape(1, -1)
    out = pl.pallas_call(
        _fused_kernel,
        grid=(_NT,),
        in_specs=[
            pl.BlockSpec(memory_space=pltpu.SMEM),
            pl.BlockSpec((_T, 1), lambda i: (i, 0)),
            pl.BlockSpec((_T, 4), lambda i: (i, 0)),
            pl.BlockSpec((4, 64), lambda i: (0, 0)),
            pl.BlockSpec((1, 64), lambda i: (0, 0)),
            pl.BlockSpec((64, 64), lambda i: (0, 0)),
            pl.BlockSpec((1, 64), lambda i: (0, 0)),
            pl.BlockSpec((64, 64), lambda i: (0, 0)),
            pl.BlockSpec((1, 64), lambda i: (0, 0)),
            pl.BlockSpec((_D * _D, _OUT), lambda i: (0, 0)),
            pl.BlockSpec((1, _OUT), lambda i: (0, 0)),
        ],
        out_specs=pl.BlockSpec((_B, _OUT), lambda i: (0, 0)),
        out_shape=jax.ShapeDtypeStruct((_B, _OUT), jnp.float32),
        scratch_shapes=[
            pltpu.VMEM((_B, _OUT), jnp.float32),
            pltpu.VMEM((1, _B), jnp.float32),
        ],
    )(bounds, seg.reshape(_N, 1), points, W1, b1.reshape(1, -1), W2,
      b2.reshape(1, -1), W3p, b3p, Wh, bh.reshape(1, -1))
    return out


# speedup vs baseline: 1.5756x; 1.4185x over previous
import jax
import jax.numpy as jnp
from jax.experimental import pallas as pl
from jax.experimental.pallas import tpu as pltpu

_N = 32768
_B = 16
_D = 16
_T = 2048
_NT = _N // _T
_OUT = 256


def _fused_kernel(bounds_ref, seg_ref, pts_ref, W1_ref, b1_ref, W2_ref,
                  b2_ref, W3_ref, b3_ref, Wh_ref, bh_ref, y_ref, acc_ref,
                  cnt_ref):
    i = pl.program_id(0)

    @pl.when(i == 0)
    def _():
        acc_ref[...] = jnp.zeros_like(acc_ref)
        cnt_ref[...] = jnp.zeros_like(cnt_ref)

    x = pts_ref[...]
    h = jnp.dot(x, W1_ref[...], preferred_element_type=jnp.float32) + b1_ref[...]
    h = jnp.maximum(h, 0.0)
    h = jnp.dot(h, W2_ref[...], preferred_element_type=jnp.float32) + b2_ref[...]
    h = jnp.maximum(h, 0.0)
    h = jnp.dot(h, W3_ref[...], preferred_element_type=jnp.float32) + b3_ref[...]
    f = h[:, :_D]
    ft = jnp.transpose(f, (1, 0))

    seg = seg_ref[0]
    bidx = jax.lax.broadcasted_iota(jnp.int32, (_B, 1), 0)
    mt = (seg == bidx).astype(jnp.float32)
    cnt_ref[...] += jnp.sum(mt, axis=1, keepdims=True)

    lo = bounds_ref[i, 0]
    hi = bounds_ref[i, 1]

    for b in range(_B):
        @pl.when(jnp.logical_and(b >= lo, b <= hi))
        def _(b=b):
            ftm = ft * mt[b:b + 1, :]
            c = jax.lax.dot_general(
                ftm, f, (((1,), (0,)), ((), ())),
                preferred_element_type=jnp.float32)
            cflat = jnp.concatenate(
                [c[k:k + 1, :] for k in range(_D)], axis=1)
            acc_ref[b:b + 1, :] += cflat

    @pl.when(i == _NT - 1)
    def _():
        maxc = jnp.max(cnt_ref[...])
        cov = acc_ref[...] / maxc
        p = jnp.sign(cov) * jnp.sqrt(jnp.abs(cov) + 1e-12)
        y = jnp.dot(p, Wh_ref[...], preferred_element_type=jnp.float32) + bh_ref[...]
        nrm = jnp.sqrt(jnp.sum(y * y, axis=1, keepdims=True))
        y_ref[...] = y / (nrm + 1e-12)


def kernel(points, segment_ids, W1, b1, W2, b2, W3, b3, Wh, bh):
    seg = segment_ids.astype(jnp.int32)
    bounds = jnp.stack([seg[::_T], seg[_T - 1::_T]], axis=1)
    W3p = jnp.pad(W3, ((0, 0), (0, 64 - _D)))
    b3p = jnp.pad(b3, (0, 64 - _D)).reshape(1, -1)
    out = pl.pallas_call(
        _fused_kernel,
        grid=(_NT,),
        in_specs=[
            pl.BlockSpec(memory_space=pltpu.SMEM),
            pl.BlockSpec((1, 1, _T), lambda i: (i, 0, 0)),
            pl.BlockSpec((_T, 4), lambda i: (i, 0)),
            pl.BlockSpec((4, 64), lambda i: (0, 0)),
            pl.BlockSpec((1, 64), lambda i: (0, 0)),
            pl.BlockSpec((64, 64), lambda i: (0, 0)),
            pl.BlockSpec((1, 64), lambda i: (0, 0)),
            pl.BlockSpec((64, 64), lambda i: (0, 0)),
            pl.BlockSpec((1, 64), lambda i: (0, 0)),
            pl.BlockSpec((_D * _D, _OUT), lambda i: (0, 0)),
            pl.BlockSpec((1, _OUT), lambda i: (0, 0)),
        ],
        out_specs=pl.BlockSpec((_B, _OUT), lambda i: (0, 0)),
        out_shape=jax.ShapeDtypeStruct((_B, _OUT), jnp.float32),
        scratch_shapes=[
            pltpu.VMEM((_B, _OUT), jnp.float32),
            pltpu.VMEM((_B, 1), jnp.float32),
        ],
    )(bounds, seg.reshape(_NT, 1, _T), points, W1, b1.reshape(1, -1), W2,
      b2.reshape(1, -1), W3p, b3p, Wh, bh.reshape(1, -1))
    return out


# speedup vs baseline: 1.5814x; 1.0037x over previous
import jax
import jax.numpy as jnp
from jax.experimental import pallas as pl
from jax.experimental.pallas import tpu as pltpu

_N = 32768
_B = 16
_D = 16
_T = 2048
_NT = _N // _T
_OUT = 256


def _fused_kernel(bounds_ref, seg_ref, pts_ref, W1_ref, b1_ref, W2_ref,
                  b2_ref, W3_ref, b3_ref, Wh_ref, bh_ref, y_ref, acc_ref,
                  cnt_ref):
    i = pl.program_id(0)

    @pl.when(i == 0)
    def _():
        acc_ref[...] = jnp.zeros_like(acc_ref)
        cnt_ref[...] = jnp.zeros_like(cnt_ref)

    x = pts_ref[...]
    h = jnp.dot(x, W1_ref[...], preferred_element_type=jnp.float32) + b1_ref[...]
    h = jnp.maximum(h, 0.0)
    h = jnp.dot(h, W2_ref[...], preferred_element_type=jnp.float32) + b2_ref[...]
    h = jnp.maximum(h, 0.0)
    h = jnp.dot(h, W3_ref[...], preferred_element_type=jnp.float32) + b3_ref[...]
    f = h[:, :_D]
    ft = jnp.transpose(f, (1, 0))

    seg = seg_ref[0]
    bidx = jax.lax.broadcasted_iota(jnp.int32, (_B, 1), 0)
    mt = (seg == bidx).astype(jnp.float32)
    cnt_ref[...] += jnp.sum(mt, axis=1, keepdims=True)

    lo = bounds_ref[i, 0]
    hi = bounds_ref[i, 1]
    rare = (hi - lo) >= 2

    def _flat(c):
        return jnp.concatenate([c[k:k + 1, :] for k in range(_D)], axis=1)

    g_full = jax.lax.dot_general(ft, f, (((1,), (0,)), ((), ())),
                                 preferred_element_type=jnp.float32)
    m_lo = (seg == lo).astype(jnp.float32)
    g_lo = jax.lax.dot_general(ft * m_lo, f, (((1,), (0,)), ((), ())),
                               preferred_element_type=jnp.float32)
    zero = jnp.where(rare, 0.0, 1.0)
    acc_ref[pl.ds(lo, 1), :] += _flat(g_lo) * zero
    acc_ref[pl.ds(hi, 1), :] += _flat(g_full - g_lo) * zero

    @pl.when(rare)
    def _():
        for b in range(_B):
            ftm = ft * mt[b:b + 1, :]
            c = jax.lax.dot_general(ftm, f, (((1,), (0,)), ((), ())),
                                    preferred_element_type=jnp.float32)
            acc_ref[b:b + 1, :] += _flat(c)

    @pl.when(i == _NT - 1)
    def _():
        maxc = jnp.max(cnt_ref[...])
        cov = acc_ref[...] / maxc
        p = jnp.sign(cov) * jnp.sqrt(jnp.abs(cov) + 1e-12)
        y = jnp.dot(p, Wh_ref[...], preferred_element_type=jnp.float32) + bh_ref[...]
        nrm = jnp.sqrt(jnp.sum(y * y, axis=1, keepdims=True))
        y_ref[...] = y / (nrm + 1e-12)


def kernel(points, segment_ids, W1, b1, W2, b2, W3, b3, Wh, bh):
    seg = segment_ids.astype(jnp.int32)
    bounds = jnp.stack([seg[::_T], seg[_T - 1::_T]], axis=1)
    W3p = jnp.pad(W3, ((0, 0), (0, 64 - _D)))
    b3p = jnp.pad(b3, (0, 64 - _D)).reshape(1, -1)
    out = pl.pallas_call(
        _fused_kernel,
        grid=(_NT,),
        in_specs=[
            pl.BlockSpec(memory_space=pltpu.SMEM),
            pl.BlockSpec((1, 1, _T), lambda i: (i, 0, 0)),
            pl.BlockSpec((_T, 4), lambda i: (i, 0)),
            pl.BlockSpec((4, 64), lambda i: (0, 0)),
            pl.BlockSpec((1, 64), lambda i: (0, 0)),
            pl.BlockSpec((64, 64), lambda i: (0, 0)),
            pl.BlockSpec((1, 64), lambda i: (0, 0)),
            pl.BlockSpec((64, 64), lambda i: (0, 0)),
            pl.BlockSpec((1, 64), lambda i: (0, 0)),
            pl.BlockSpec((_D * _D, _OUT), lambda i: (0, 0)),
            pl.BlockSpec((1, _OUT), lambda i: (0, 0)),
        ],
        out_specs=pl.BlockSpec((_B, _OUT), lambda i: (0, 0)),
        out_shape=jax.ShapeDtypeStruct((_B, _OUT), jnp.float32),
        scratch_shapes=[
            pltpu.VMEM((_B, _OUT), jnp.float32),
            pltpu.VMEM((_B, 1), jnp.float32),
        ],
    )(bounds, seg.reshape(_NT, 1, _T), points, W1, b1.reshape(1, -1), W2,
      b2.reshape(1, -1), W3p, b3p, Wh, bh.reshape(1, -1))
    return out


# speedup vs baseline: 1.6371x; 1.0352x over previous
import jax
import jax.numpy as jnp
from jax.experimental import pallas as pl
from jax.experimental.pallas import tpu as pltpu

_N = 32768
_B = 16
_D = 16
_T = 2048
_NT = _N // _T
_OUT = 256


def _fused_kernel(bounds_ref, seg_ref, pts_ref, W1_ref, b1_ref, W2_ref,
                  b2_ref, W3_ref, b3_ref, Wh_ref, bh_ref, y_ref, acc_ref,
                  cnt_ref):
    i = pl.program_id(0)

    @pl.when(i == 0)
    def _():
        acc_ref[...] = jnp.zeros_like(acc_ref)
        cnt_ref[...] = jnp.zeros_like(cnt_ref)

    x = pts_ref[...]
    h = jnp.dot(x, W1_ref[...], preferred_element_type=jnp.float32) + b1_ref[...]
    h = jnp.maximum(h, 0.0)
    h = jnp.dot(h, W2_ref[...], preferred_element_type=jnp.float32) + b2_ref[...]
    h = jnp.maximum(h, 0.0)
    h = jnp.dot(h, W3_ref[...], preferred_element_type=jnp.float32) + b3_ref[...]
    f = h[:, :_D].astype(jnp.bfloat16)
    ft = jnp.transpose(f, (1, 0))

    seg = seg_ref[0]
    bidx = jax.lax.broadcasted_iota(jnp.int32, (_B, 1), 0)
    mt = (seg == bidx).astype(jnp.bfloat16)
    cnt_ref[...] += jnp.sum(mt.astype(jnp.float32), axis=1, keepdims=True)

    lo = bounds_ref[i, 0]
    hi = bounds_ref[i, 1]
    rare = (hi - lo) >= 2

    g_full = jax.lax.dot_general(ft, f, (((1,), (0,)), ((), ())),
                                 preferred_element_type=jnp.float32)
    m_lo = (seg == lo).astype(jnp.bfloat16)
    g_lo = jax.lax.dot_general(ft * m_lo, f, (((1,), (0,)), ((), ())),
                               preferred_element_type=jnp.float32)
    zero = jnp.where(rare, 0.0, 1.0)
    acc_ref[pl.ds(lo * _D, _D), :] += g_lo * zero
    acc_ref[pl.ds(hi * _D, _D), :] += (g_full - g_lo) * zero

    @pl.when(rare)
    def _():
        for b in range(_B):
            ftm = ft * mt[b:b + 1, :]
            c = jax.lax.dot_general(ftm, f, (((1,), (0,)), ((), ())),
                                    preferred_element_type=jnp.float32)
            acc_ref[b * _D:(b + 1) * _D, :] += c

    @pl.when(i == _NT - 1)
    def _():
        maxc = jnp.max(cnt_ref[...])
        cov = acc_ref[...] / maxc
        p = jnp.sign(cov) * jnp.sqrt(jnp.abs(cov) + 1e-12)
        pflat = jnp.concatenate(
            [jnp.concatenate([p[_D * b + k:_D * b + k + 1, :]
                              for k in range(_D)], axis=1)
             for b in range(_B)], axis=0)
        y = jnp.dot(pflat, Wh_ref[...], preferred_element_type=jnp.float32) + bh_ref[...]
        nrm = jnp.sqrt(jnp.sum(y * y, axis=1, keepdims=True))
        y_ref[...] = y / (nrm + 1e-12)


def kernel(points, segment_ids, W1, b1, W2, b2, W3, b3, Wh, bh):
    seg = segment_ids.astype(jnp.int32)
    bounds = jnp.stack([seg[::_T], seg[_T - 1::_T]], axis=1)
    W3p = jnp.pad(W3, ((0, 0), (0, 64 - _D)))
    b3p = jnp.pad(b3, (0, 64 - _D)).reshape(1, -1)
    out = pl.pallas_call(
        _fused_kernel,
        grid=(_NT,),
        in_specs=[
            pl.BlockSpec(memory_space=pltpu.SMEM),
            pl.BlockSpec((1, 1, _T), lambda i: (i, 0, 0)),
            pl.BlockSpec((_T, 4), lambda i: (i, 0)),
            pl.BlockSpec((4, 64), lambda i: (0, 0)),
            pl.BlockSpec((1, 64), lambda i: (0, 0)),
            pl.BlockSpec((64, 64), lambda i: (0, 0)),
            pl.BlockSpec((1, 64), lambda i: (0, 0)),
            pl.BlockSpec((64, 64), lambda i: (0, 0)),
            pl.BlockSpec((1, 64), lambda i: (0, 0)),
            pl.BlockSpec((_D * _D, _OUT), lambda i: (0, 0)),
            pl.BlockSpec((1, _OUT), lambda i: (0, 0)),
        ],
        out_specs=pl.BlockSpec((_B, _OUT), lambda i: (0, 0)),
        out_shape=jax.ShapeDtypeStruct((_B, _OUT), jnp.float32),
        scratch_shapes=[
            pltpu.VMEM((_B * _D, _D), jnp.float32),
            pltpu.VMEM((_B, 1), jnp.float32),
        ],
    )(bounds, seg.reshape(_NT, 1, _T), points, W1, b1.reshape(1, -1), W2,
      b2.reshape(1, -1), W3p, b3p, Wh, bh.reshape(1, -1))
    return out


# speedup vs baseline: 2.6370x; 1.6108x over previous
import jax
import jax.numpy as jnp
from jax.experimental import pallas as pl
from jax.experimental.pallas import tpu as pltpu

_N = 32768
_B = 16
_D = 16
_T = 2048
_NT = _N // _T
_OUT = 256


def _fused_kernel(bounds_ref, seg_ref, xa_ref, W1_ref, W2_ref, W3_ref,
                  Wh_ref, bh_ref, y_ref, acc_ref, cnt_ref):
    i = pl.program_id(0)

    @pl.when(i == 0)
    def _():
        acc_ref[...] = jnp.zeros_like(acc_ref)
        cnt_ref[...] = jnp.zeros_like(cnt_ref)

    ones = jnp.ones((1, _T), jnp.float32)
    xa = xa_ref[...]
    h = jnp.maximum(
        jnp.dot(W1_ref[...], xa, preferred_element_type=jnp.float32), 0.0)
    h = jnp.concatenate([h, ones], axis=0)
    h = jnp.maximum(
        jnp.dot(W2_ref[...], h, preferred_element_type=jnp.float32), 0.0)
    h = jnp.concatenate([h, ones], axis=0)
    ft = jnp.dot(W3_ref[...], h,
                 preferred_element_type=jnp.float32).astype(jnp.bfloat16)
    f = jnp.transpose(ft, (1, 0))

    seg = seg_ref[0]
    bidx = jax.lax.broadcasted_iota(jnp.int32, (_B, 1), 0)
    mt = (seg == bidx).astype(jnp.bfloat16)
    cnt_ref[...] += jnp.sum(mt.astype(jnp.float32), axis=1, keepdims=True)

    lo = bounds_ref[i, 0]
    hi = bounds_ref[i, 1]
    rare = (hi - lo) >= 2

    g_full = jax.lax.dot_general(ft, f, (((1,), (0,)), ((), ())),
                                 preferred_element_type=jnp.float32)
    m_lo = (seg == lo).astype(jnp.bfloat16)
    g_lo = jax.lax.dot_general(ft * m_lo, f, (((1,), (0,)), ((), ())),
                               preferred_element_type=jnp.float32)
    zero = jnp.where(rare, 0.0, 1.0)
    acc_ref[pl.ds(lo * _D, _D), :] += g_lo * zero
    acc_ref[pl.ds(hi * _D, _D), :] += (g_full - g_lo) * zero

    @pl.when(rare)
    def _():
        for b in range(_B):
            ftm = ft * mt[b:b + 1, :]
            c = jax.lax.dot_general(ftm, f, (((1,), (0,)), ((), ())),
                                    preferred_element_type=jnp.float32)
            acc_ref[b * _D:(b + 1) * _D, :] += c

    @pl.when(i == _NT - 1)
    def _():
        maxc = jnp.max(cnt_ref[...])
        cov = acc_ref[...] / maxc
        p = jnp.sign(cov) * jnp.sqrt(jnp.abs(cov) + 1e-12)
        pflat = jnp.concatenate(
            [jnp.concatenate([p[_D * b + k:_D * b + k + 1, :]
                              for k in range(_D)], axis=1)
             for b in range(_B)], axis=0)
        y = jnp.dot(pflat, Wh_ref[...], preferred_element_type=jnp.float32) + bh_ref[...]
        nrm = jnp.sqrt(jnp.sum(y * y, axis=1, keepdims=True))
        y_ref[...] = y / (nrm + 1e-12)


def kernel(points, segment_ids, W1, b1, W2, b2, W3, b3, Wh, bh):
    seg = segment_ids.astype(jnp.int32)
    bounds = jnp.stack([seg[::_T], seg[_T - 1::_T]], axis=1)
    xa = jnp.concatenate(
        [points.T, jnp.ones((1, _N), jnp.float32)], axis=0)
    W1a = jnp.concatenate([W1, b1[None, :]], axis=0).T
    W2a = jnp.concatenate([W2, b2[None, :]], axis=0).T
    W3a = jnp.concatenate([W3, b3[None, :]], axis=0).T
    out = pl.pallas_call(
        _fused_kernel,
        grid=(_NT,),
        in_specs=[
            pl.BlockSpec(memory_space=pltpu.SMEM),
            pl.BlockSpec((1, 1, _T), lambda i: (i, 0, 0)),
            pl.BlockSpec((5, _T), lambda i: (0, i)),
            pl.BlockSpec((64, 5), lambda i: (0, 0)),
            pl.BlockSpec((64, 65), lambda i: (0, 0)),
            pl.BlockSpec((_D, 65), lambda i: (0, 0)),
            pl.BlockSpec((_D * _D, _OUT), lambda i: (0, 0)),
            pl.BlockSpec((1, _OUT), lambda i: (0, 0)),
        ],
        out_specs=pl.BlockSpec((_B, _OUT), lambda i: (0, 0)),
        out_shape=jax.ShapeDtypeStruct((_B, _OUT), jnp.float32),
        scratch_shapes=[
            pltpu.VMEM((_B * _D, _D), jnp.float32),
            pltpu.VMEM((_B, 1), jnp.float32),
        ],
    )(bounds, seg.reshape(_NT, 1, _T), xa, W1a, W2a, W3a, Wh,
      bh.reshape(1, -1))
    return out


# speedup vs baseline: 3.2268x; 1.2236x over previous
import jax
import jax.numpy as jnp
from jax.experimental import pallas as pl
from jax.experimental.pallas import tpu as pltpu

_N = 32768
_B = 16
_D = 16
_T = 4096
_NT = _N // _T
_OUT = 256


def _fused_kernel(bounds_ref, seg_ref, xa_ref, W1_ref, W2_ref, W3_ref,
                  Wh_ref, bh_ref, y_ref, acc_ref, cnt_ref):
    i = pl.program_id(0)

    @pl.when(i == 0)
    def _():
        acc_ref[...] = jnp.zeros_like(acc_ref)
        cnt_ref[...] = jnp.zeros_like(cnt_ref)

    ones = jnp.ones((1, _T), jnp.float32)
    xa = xa_ref[...]
    h = jnp.maximum(
        jnp.dot(W1_ref[...], xa, preferred_element_type=jnp.float32), 0.0)
    h = jnp.concatenate([h, ones], axis=0)
    h = jnp.maximum(
        jnp.dot(W2_ref[...], h, preferred_element_type=jnp.float32), 0.0)
    h = jnp.concatenate([h, ones], axis=0)
    ft = jnp.dot(W3_ref[...], h,
                 preferred_element_type=jnp.float32).astype(jnp.bfloat16)
    f = jnp.transpose(ft, (1, 0))

    seg = seg_ref[0]
    bidx = jax.lax.broadcasted_iota(jnp.int32, (_B, 1), 0)
    mt = (seg == bidx).astype(jnp.bfloat16)
    cnt_ref[...] += jnp.sum(mt.astype(jnp.float32), axis=1, keepdims=True)

    lo = bounds_ref[i, 0]
    hi = bounds_ref[i, 1]
    mid = jnp.minimum(lo + 1, _B - 1)
    rare = (hi - lo) >= 3

    g_full = jax.lax.dot_general(ft, f, (((1,), (0,)), ((), ())),
                                 preferred_element_type=jnp.float32)
    m_lo = (seg == lo).astype(jnp.bfloat16)
    g_lo = jax.lax.dot_general(ft * m_lo, f, (((1,), (0,)), ((), ())),
                               preferred_element_type=jnp.float32)
    m_mid = (seg == mid).astype(jnp.bfloat16)
    g_mid = jax.lax.dot_general(ft * m_mid, f, (((1,), (0,)), ((), ())),
                                preferred_element_type=jnp.float32)
    zero = jnp.where(rare, 0.0, 1.0)
    acc_ref[pl.ds(lo * _D, _D), :] += g_lo * zero
    acc_ref[pl.ds(mid * _D, _D), :] += g_mid * zero
    acc_ref[pl.ds(hi * _D, _D), :] += (g_full - g_lo - g_mid) * zero

    @pl.when(rare)
    def _():
        for b in range(_B):
            ftm = ft * mt[b:b + 1, :]
            c = jax.lax.dot_general(ftm, f, (((1,), (0,)), ((), ())),
                                    preferred_element_type=jnp.float32)
            acc_ref[b * _D:(b + 1) * _D, :] += c

    @pl.when(i == _NT - 1)
    def _():
        maxc = jnp.max(cnt_ref[...])
        cov = acc_ref[...] / maxc
        p = jnp.sign(cov) * jnp.sqrt(jnp.abs(cov) + 1e-12)
        pflat = jnp.concatenate(
            [jnp.concatenate([p[_D * b + k:_D * b + k + 1, :]
                              for k in range(_D)], axis=1)
             for b in range(_B)], axis=0)
        y = jnp.dot(pflat, Wh_ref[...], preferred_element_type=jnp.float32) + bh_ref[...]
        nrm = jnp.sqrt(jnp.sum(y * y, axis=1, keepdims=True))
        y_ref[...] = y / (nrm + 1e-12)


def kernel(points, segment_ids, W1, b1, W2, b2, W3, b3, Wh, bh):
    seg = segment_ids.astype(jnp.int32)
    bounds = jnp.stack([seg[::_T], seg[_T - 1::_T]], axis=1)
    xa = jnp.concatenate(
        [points.T, jnp.ones((1, _N), jnp.float32)], axis=0)
    W1a = jnp.concatenate([W1, b1[None, :]], axis=0).T
    W2a = jnp.concatenate([W2, b2[None, :]], axis=0).T
    W3a = jnp.concatenate([W3, b3[None, :]], axis=0).T
    out = pl.pallas_call(
        _fused_kernel,
        grid=(_NT,),
        in_specs=[
            pl.BlockSpec(memory_space=pltpu.SMEM),
            pl.BlockSpec((1, 1, _T), lambda i: (i, 0, 0)),
            pl.BlockSpec((5, _T), lambda i: (0, i)),
            pl.BlockSpec((64, 5), lambda i: (0, 0)),
            pl.BlockSpec((64, 65), lambda i: (0, 0)),
            pl.BlockSpec((_D, 65), lambda i: (0, 0)),
            pl.BlockSpec((_D * _D, _OUT), lambda i: (0, 0)),
            pl.BlockSpec((1, _OUT), lambda i: (0, 0)),
        ],
        out_specs=pl.BlockSpec((_B, _OUT), lambda i: (0, 0)),
        out_shape=jax.ShapeDtypeStruct((_B, _OUT), jnp.float32),
        scratch_shapes=[
            pltpu.VMEM((_B * _D, _D), jnp.float32),
            pltpu.VMEM((_B, 1), jnp.float32),
        ],
    )(bounds, seg.reshape(_NT, 1, _T), xa, W1a, W2a, W3a, Wh,
      bh.reshape(1, -1))
    return out
